# baseline (speedup 1.0000x reference)
import functools
import math

import jax
import jax.numpy as jnp
import numpy as np
from jax import lax
from jax.experimental import pallas as pl
from jax.experimental.pallas import tpu as pltpu
from jax.experimental.pallas import tpu_sc as plsc

N_NODES = 50000
N_EDGES = 1600000
IN_F = 128
H_F = 32
DEG_W = 16
D_POLY = 2

NC, NS = 2, 16
NW = NC * NS
CHUNK = 125
NCHUNKS = N_EDGES // CHUNK
CPW = NCHUNKS // NW
NPAD = 50048
RPS = NPAD // NS
GRP = 16


def _theta_coeffs(d):
    thetas = []
    for i in range(d + 1):
        p1 = np.zeros(i + 1)
        p1[i] = 0.5 ** i
        m = d - i
        p2 = np.array([math.comb(m, k) * (-0.5) ** k for k in range(m + 1)])
        poly = np.convolve(p1, p2)
        beta = math.gamma(i + 1) * math.gamma(d + 1 - i) / math.gamma(d + 2)
        thetas.append(poly / beta)
    return np.stack(thetas)


_COEFF = _theta_coeffs(D_POLY)


@functools.lru_cache(maxsize=None)
def _sc_kernels():
    mesh = plsc.VectorSubcoreMesh(
        core_axis_name="c", subcore_axis_name="s",
        num_cores=NC, num_subcores=NS)

    params = pltpu.CompilerParams(use_tc_tiling_on_sc=False)

    @functools.partial(
        pl.kernel,
        mesh=mesh,
        compiler_params=params,
        out_type=jax.ShapeDtypeStruct((NC, NPAD, DEG_W), jnp.float32),
        scratch_types=[
            pltpu.VMEM((GRP, CHUNK), jnp.int32),
            pltpu.VMEM((CHUNK, DEG_W), jnp.float32),
            pltpu.VMEM_SHARED((NPAD, DEG_W), jnp.float32),
        ],
    )
    def _sc_degree(dstr_hbm, ones_hbm, zeros_hbm, out_hbm, idx_d, ones_v, acc):
        c = lax.axis_index("c")
        s = lax.axis_index("s")
        wid = c * NS + s
        start = wid * CPW
        pltpu.sync_copy(zeros_hbm, acc.at[pl.ds(s * RPS, RPS)])
        pltpu.sync_copy(ones_hbm, ones_v)
        plsc.subcore_barrier()

        def group(g, carry):
            pltpu.sync_copy(dstr_hbm.at[pl.ds(start + g * GRP, GRP)], idx_d)

            def body(j, carry2):
                pltpu.sync_copy(ones_v, acc.at[idx_d.at[j]], add=True)
                return carry2

            lax.fori_loop(0, GRP, body, 0)
            return carry

        lax.fori_loop(0, CPW // GRP, group, 0)
        plsc.subcore_barrier()
        pltpu.sync_copy(acc.at[pl.ds(s * RPS, RPS)],
                        out_hbm.at[c, pl.ds(s * RPS, RPS)])

    @functools.partial(
        pl.kernel,
        mesh=mesh,
        compiler_params=params,
        out_type=jax.ShapeDtypeStruct((NC, NPAD, H_F), jnp.float32),
        scratch_types=[
            pltpu.VMEM((GRP, CHUNK), jnp.int32),
            pltpu.VMEM((GRP, CHUNK), jnp.int32),
            pltpu.VMEM((CHUNK, H_F), jnp.float32),
            pltpu.VMEM((CHUNK, H_F), jnp.float32),
            pltpu.VMEM_SHARED((NPAD, H_F), jnp.float32),
            pltpu.SemaphoreType.DMA,
            pltpu.SemaphoreType.DMA,
        ],
    )
    def _sc_segsum(z_hbm, srcr_hbm, dstr_hbm, zeros_hbm, out_hbm,
                   idx_s, idx_d, rows_a, rows_b, acc, sem_a, sem_b):
        c = lax.axis_index("c")
        s = lax.axis_index("s")
        wid = c * NS + s
        start = wid * CPW
        pltpu.sync_copy(zeros_hbm, acc.at[pl.ds(s * RPS, RPS)])
        plsc.subcore_barrier()

        def group(g, carry):
            pltpu.sync_copy(srcr_hbm.at[pl.ds(start + g * GRP, GRP)], idx_s)
            pltpu.sync_copy(dstr_hbm.at[pl.ds(start + g * GRP, GRP)], idx_d)
            pltpu.async_copy(z_hbm.at[idx_s.at[0]], rows_a, sem_a)

            def body(j2, carry2):
                j = 2 * j2
                pltpu.async_copy(z_hbm.at[idx_s.at[j + 1]], rows_b, sem_b)
                pltpu.make_async_copy(z_hbm.at[idx_s.at[j]],
                                      rows_a, sem_a).wait()
                pltpu.sync_copy(rows_a, acc.at[idx_d.at[j]], add=True)

                @pl.when(j2 < GRP // 2 - 1)
                def _():
                    pltpu.async_copy(z_hbm.at[idx_s.at[j + 2]], rows_a, sem_a)

                pltpu.make_async_copy(z_hbm.at[idx_s.at[j + 1]],
                                      rows_b, sem_b).wait()
                pltpu.sync_copy(rows_b, acc.at[idx_d.at[j + 1]], add=True)
                return carry2

            lax.fori_loop(0, GRP // 2, body, 0)
            return carry

        lax.fori_loop(0, CPW // GRP, group, 0)
        plsc.subcore_barrier()
        pltpu.sync_copy(acc.at[pl.ds(s * RPS, RPS)],
                        out_hbm.at[c, pl.ds(s * RPS, RPS)])

    return _sc_degree, _sc_segsum


_ROWS_BLK = 2000
_N_BLKS = N_NODES // _ROWS_BLK


def _lin1_body(x_ref, degp_ref, w1t_ref, b1_ref, h_ref, z0_ref, dinv_ref):
    x = x_ref[...]
    h = jnp.dot(x, w1t_ref[...], preferred_element_type=jnp.float32,
                precision=lax.Precision.HIGHEST)
    h = jnp.maximum(h + b1_ref[...], 0.0)
    deg = degp_ref[0, :, 0:1] + degp_ref[1, :, 0:1]
    dinv = lax.rsqrt(jnp.maximum(deg, 1.0))
    h_ref[...] = h
    z0_ref[...] = h * dinv
    dinv_ref[...] = dinv


def _tc_lin1(x, degp, w1t, b1r):
    f32 = jnp.float32
    return pl.pallas_call(
        _lin1_body,
        grid=(_N_BLKS,),
        in_specs=[
            pl.BlockSpec((_ROWS_BLK, IN_F), lambda i: (i, 0)),
            pl.BlockSpec((NC, _ROWS_BLK, DEG_W), lambda i: (0, i, 0)),
            pl.BlockSpec((IN_F, H_F), lambda i: (0, 0)),
            pl.BlockSpec((1, H_F), lambda i: (0, 0)),
        ],
        out_specs=[
            pl.BlockSpec((_ROWS_BLK, H_F), lambda i: (i, 0)),
            pl.BlockSpec((_ROWS_BLK, H_F), lambda i: (i, 0)),
            pl.BlockSpec((_ROWS_BLK, 1), lambda i: (i, 0)),
        ],
        out_shape=[
            jax.ShapeDtypeStruct((N_NODES, H_F), f32),
            jax.ShapeDtypeStruct((N_NODES, H_F), f32),
            jax.ShapeDtypeStruct((N_NODES, 1), f32),
        ],
    )(x, degp, w1t, b1r)


def _update_body(h_ref, aggp_ref, dinv_ref, f1_ref, z1_ref):
    agg = aggp_ref[0] + aggp_ref[1]
    dinv = dinv_ref[...]
    f1 = h_ref[...] - agg * dinv
    f1_ref[...] = f1
    z1_ref[...] = f1 * dinv


def _tc_update(h, aggp, dinv):
    f32 = jnp.float32
    return pl.pallas_call(
        _update_body,
        grid=(_N_BLKS,),
        in_specs=[
            pl.BlockSpec((_ROWS_BLK, H_F), lambda i: (i, 0)),
            pl.BlockSpec((NC, _ROWS_BLK, H_F), lambda i: (0, i, 0)),
            pl.BlockSpec((_ROWS_BLK, 1), lambda i: (i, 0)),
        ],
        out_specs=[
            pl.BlockSpec((_ROWS_BLK, H_F), lambda i: (i, 0)),
            pl.BlockSpec((_ROWS_BLK, H_F), lambda i: (i, 0)),
        ],
        out_shape=[
            jax.ShapeDtypeStruct((N_NODES, H_F), f32),
            jax.ShapeDtypeStruct((N_NODES, H_F), f32),
        ],
    )(h, aggp, dinv)


def _final_body(h_ref, f1_ref, aggp_ref, dinv_ref, g_ref, b2_ref, out_ref):
    agg = aggp_ref[0] + aggp_ref[1]
    f1 = f1_ref[...]
    f2 = f1 - agg * dinv_ref[...]
    cat = jnp.concatenate([h_ref[...], f1, f2], axis=-1)
    out_ref[...] = (
        jnp.dot(cat, g_ref[...], preferred_element_type=jnp.float32,
                precision=lax.Precision.HIGHEST)
        + b2_ref[...]
    )


def _tc_final(h, f1, aggp, dinv, g, b2r):
    return pl.pallas_call(
        _final_body,
        grid=(_N_BLKS,),
        in_specs=[
            pl.BlockSpec((_ROWS_BLK, H_F), lambda i: (i, 0)),
            pl.BlockSpec((_ROWS_BLK, H_F), lambda i: (i, 0)),
            pl.BlockSpec((NC, _ROWS_BLK, H_F), lambda i: (0, i, 0)),
            pl.BlockSpec((_ROWS_BLK, 1), lambda i: (i, 0)),
            pl.BlockSpec((3 * H_F, H_F), lambda i: (0, 0)),
            pl.BlockSpec((1, H_F), lambda i: (0, 0)),
        ],
        out_specs=pl.BlockSpec((_ROWS_BLK, H_F), lambda i: (i, 0)),
        out_shape=jax.ShapeDtypeStruct((N_NODES, H_F), jnp.float32),
    )(h, f1, aggp, dinv, g, b2r)


def kernel(features, edge_index, W1, b1, W2, b2):
    f32 = jnp.float32
    srcr = edge_index[0].reshape(NCHUNKS, CHUNK)
    dstr = edge_index[1].reshape(NCHUNKS, CHUNK)

    ones_deg = jnp.ones((CHUNK, DEG_W), f32)
    zeros_deg = jnp.zeros((RPS, DEG_W), f32)
    zeros_f = jnp.zeros((RPS, H_F), f32)

    w1t = W1.T
    b1r = b1.reshape(1, H_F)
    w2b = W2.T.reshape(D_POLY + 1, H_F, H_F)
    coeff = jnp.asarray(_COEFF, f32)
    g = jnp.tensordot(coeff, w2b, axes=((0,), (0,)))
    g = g.reshape((D_POLY + 1) * H_F, H_F)
    b2r = b2.reshape(1, H_F)

    sc_degree, sc_segsum = _sc_kernels()
    degp = sc_degree(dstr, ones_deg, zeros_deg)
    h, z0, dinv = _tc_lin1(features, degp, w1t, b1r)
    agg1 = sc_segsum(z0, srcr, dstr, zeros_f)
    f1, z1 = _tc_update(h, agg1, dinv)
    agg2 = sc_segsum(z1, srcr, dstr, zeros_f)
    return _tc_final(h, f1, agg2, dinv, g, b2r)


# speedup vs baseline: 1.1847x; 1.1847x over previous
import functools
import math

import jax
import jax.numpy as jnp
import numpy as np
from jax import lax
from jax.experimental import pallas as pl
from jax.experimental.pallas import tpu as pltpu
from jax.experimental.pallas import tpu_sc as plsc

N_NODES = 50000
N_EDGES = 1600000
IN_F = 128
H_F = 32
DEG_W = 16
D_POLY = 2

NC, NS = 2, 16
NW = NC * NS
CHUNK = 125
NCHUNKS = N_EDGES // CHUNK
CPW = NCHUNKS // NW
NPAD = 50048
RPS = NPAD // NS
GRP = 16


def _theta_coeffs(d):
    thetas = []
    for i in range(d + 1):
        p1 = np.zeros(i + 1)
        p1[i] = 0.5 ** i
        m = d - i
        p2 = np.array([math.comb(m, k) * (-0.5) ** k for k in range(m + 1)])
        poly = np.convolve(p1, p2)
        beta = math.gamma(i + 1) * math.gamma(d + 1 - i) / math.gamma(d + 2)
        thetas.append(poly / beta)
    return np.stack(thetas)


_COEFF = _theta_coeffs(D_POLY)


@functools.lru_cache(maxsize=None)
def _sc_kernels():
    mesh = plsc.VectorSubcoreMesh(
        core_axis_name="c", subcore_axis_name="s",
        num_cores=NC, num_subcores=NS)

    params = pltpu.CompilerParams(use_tc_tiling_on_sc=False)

    @functools.partial(
        pl.kernel,
        mesh=mesh,
        compiler_params=params,
        out_type=jax.ShapeDtypeStruct((NC, NPAD, DEG_W), jnp.float32),
        scratch_types=[
            pltpu.VMEM((GRP, CHUNK), jnp.int32),
            pltpu.VMEM((CHUNK, DEG_W), jnp.float32),
            pltpu.VMEM_SHARED((NPAD, DEG_W), jnp.float32),
            pltpu.SemaphoreType.DMA,
        ],
    )
    def _sc_degree(dstr_hbm, ones_hbm, zeros_hbm, out_hbm, idx_d, ones_v, acc,
                   dsem):
        c = lax.axis_index("c")
        s = lax.axis_index("s")
        wid = c * NS + s
        start = wid * CPW
        pltpu.sync_copy(zeros_hbm, acc.at[pl.ds(s * RPS, RPS)])
        pltpu.sync_copy(ones_hbm, ones_v)
        plsc.subcore_barrier()

        def group(g, carry):
            pltpu.sync_copy(dstr_hbm.at[pl.ds(start + g * GRP, GRP)], idx_d)
            for j in range(GRP):
                pltpu.async_copy(ones_v, acc.at[idx_d.at[j]], dsem, add=True)
            for j in range(GRP):
                pltpu.make_async_copy(ones_v, acc.at[idx_d.at[j]],
                                      dsem).wait()
            return carry

        lax.fori_loop(0, CPW // GRP, group, 0)
        plsc.subcore_barrier()
        pltpu.sync_copy(acc.at[pl.ds(s * RPS, RPS)],
                        out_hbm.at[c, pl.ds(s * RPS, RPS)])

    @functools.partial(
        pl.kernel,
        mesh=mesh,
        compiler_params=params,
        out_type=jax.ShapeDtypeStruct((NC, NPAD, H_F), jnp.float32),
        scratch_types=[
            pltpu.VMEM((GRP, CHUNK), jnp.int32),
            pltpu.VMEM((GRP, CHUNK), jnp.int32),
            [pltpu.VMEM((CHUNK, H_F), jnp.float32)] * 4,
            pltpu.VMEM_SHARED((NPAD, H_F), jnp.float32),
            [pltpu.SemaphoreType.DMA] * 4,
            [pltpu.SemaphoreType.DMA] * 4,
        ],
    )
    def _sc_segsum(z_hbm, srcr_hbm, dstr_hbm, zeros_hbm, out_hbm,
                   idx_s, idx_d, rows, acc, gsem, ssem):
        c = lax.axis_index("c")
        s = lax.axis_index("s")
        wid = c * NS + s
        start = wid * CPW
        pltpu.sync_copy(zeros_hbm, acc.at[pl.ds(s * RPS, RPS)])
        plsc.subcore_barrier()

        def group(g, carry):
            pltpu.sync_copy(srcr_hbm.at[pl.ds(start + g * GRP, GRP)], idx_s)
            pltpu.sync_copy(dstr_hbm.at[pl.ds(start + g * GRP, GRP)], idx_d)
            pltpu.async_copy(z_hbm.at[idx_s.at[0]], rows[0], gsem[0])
            pltpu.async_copy(z_hbm.at[idx_s.at[1]], rows[1], gsem[1])
            for j in range(GRP):
                b = j % 4
                if j + 2 < GRP:
                    b2 = (j + 2) % 4
                    if j >= 2:
                        pltpu.make_async_copy(
                            rows[b2], acc.at[idx_d.at[j - 2]],
                            ssem[b2]).wait()
                    pltpu.async_copy(z_hbm.at[idx_s.at[j + 2]],
                                     rows[b2], gsem[b2])
                pltpu.make_async_copy(z_hbm.at[idx_s.at[j]],
                                      rows[b], gsem[b]).wait()
                pltpu.async_copy(rows[b], acc.at[idx_d.at[j]],
                                 ssem[b], add=True)
            for j in range(GRP - 4, GRP):
                b = j % 4
                pltpu.make_async_copy(rows[b], acc.at[idx_d.at[j]],
                                      ssem[b]).wait()
            return carry

        lax.fori_loop(0, CPW // GRP, group, 0)
        plsc.subcore_barrier()
        pltpu.sync_copy(acc.at[pl.ds(s * RPS, RPS)],
                        out_hbm.at[c, pl.ds(s * RPS, RPS)])

    return _sc_degree, _sc_segsum


_ROWS_BLK = 2000
_N_BLKS = N_NODES // _ROWS_BLK


def _lin1_body(x_ref, degp_ref, w1t_ref, b1_ref, h_ref, z0_ref, dinv_ref):
    x = x_ref[...]
    h = jnp.dot(x, w1t_ref[...], preferred_element_type=jnp.float32,
                precision=lax.Precision.HIGHEST)
    h = jnp.maximum(h + b1_ref[...], 0.0)
    deg = degp_ref[0, :, 0:1] + degp_ref[1, :, 0:1]
    dinv = lax.rsqrt(jnp.maximum(deg, 1.0))
    h_ref[...] = h
    z0_ref[...] = h * dinv
    dinv_ref[...] = dinv


def _tc_lin1(x, degp, w1t, b1r):
    f32 = jnp.float32
    return pl.pallas_call(
        _lin1_body,
        grid=(_N_BLKS,),
        in_specs=[
            pl.BlockSpec((_ROWS_BLK, IN_F), lambda i: (i, 0)),
            pl.BlockSpec((NC, _ROWS_BLK, DEG_W), lambda i: (0, i, 0)),
            pl.BlockSpec((IN_F, H_F), lambda i: (0, 0)),
            pl.BlockSpec((1, H_F), lambda i: (0, 0)),
        ],
        out_specs=[
            pl.BlockSpec((_ROWS_BLK, H_F), lambda i: (i, 0)),
            pl.BlockSpec((_ROWS_BLK, H_F), lambda i: (i, 0)),
            pl.BlockSpec((_ROWS_BLK, 1), lambda i: (i, 0)),
        ],
        out_shape=[
            jax.ShapeDtypeStruct((N_NODES, H_F), f32),
            jax.ShapeDtypeStruct((N_NODES, H_F), f32),
            jax.ShapeDtypeStruct((N_NODES, 1), f32),
        ],
    )(x, degp, w1t, b1r)


def _update_body(h_ref, aggp_ref, dinv_ref, f1_ref, z1_ref):
    agg = aggp_ref[0] + aggp_ref[1]
    dinv = dinv_ref[...]
    f1 = h_ref[...] - agg * dinv
    f1_ref[...] = f1
    z1_ref[...] = f1 * dinv


def _tc_update(h, aggp, dinv):
    f32 = jnp.float32
    return pl.pallas_call(
        _update_body,
        grid=(_N_BLKS,),
        in_specs=[
            pl.BlockSpec((_ROWS_BLK, H_F), lambda i: (i, 0)),
            pl.BlockSpec((NC, _ROWS_BLK, H_F), lambda i: (0, i, 0)),
            pl.BlockSpec((_ROWS_BLK, 1), lambda i: (i, 0)),
        ],
        out_specs=[
            pl.BlockSpec((_ROWS_BLK, H_F), lambda i: (i, 0)),
            pl.BlockSpec((_ROWS_BLK, H_F), lambda i: (i, 0)),
        ],
        out_shape=[
            jax.ShapeDtypeStruct((N_NODES, H_F), f32),
            jax.ShapeDtypeStruct((N_NODES, H_F), f32),
        ],
    )(h, aggp, dinv)


def _final_body(h_ref, f1_ref, aggp_ref, dinv_ref, g_ref, b2_ref, out_ref):
    agg = aggp_ref[0] + aggp_ref[1]
    f1 = f1_ref[...]
    f2 = f1 - agg * dinv_ref[...]
    cat = jnp.concatenate([h_ref[...], f1, f2], axis=-1)
    out_ref[...] = (
        jnp.dot(cat, g_ref[...], preferred_element_type=jnp.float32,
                precision=lax.Precision.HIGHEST)
        + b2_ref[...]
    )


def _tc_final(h, f1, aggp, dinv, g, b2r):
    return pl.pallas_call(
        _final_body,
        grid=(_N_BLKS,),
        in_specs=[
            pl.BlockSpec((_ROWS_BLK, H_F), lambda i: (i, 0)),
            pl.BlockSpec((_ROWS_BLK, H_F), lambda i: (i, 0)),
            pl.BlockSpec((NC, _ROWS_BLK, H_F), lambda i: (0, i, 0)),
            pl.BlockSpec((_ROWS_BLK, 1), lambda i: (i, 0)),
            pl.BlockSpec((3 * H_F, H_F), lambda i: (0, 0)),
            pl.BlockSpec((1, H_F), lambda i: (0, 0)),
        ],
        out_specs=pl.BlockSpec((_ROWS_BLK, H_F), lambda i: (i, 0)),
        out_shape=jax.ShapeDtypeStruct((N_NODES, H_F), jnp.float32),
    )(h, f1, aggp, dinv, g, b2r)


def kernel(features, edge_index, W1, b1, W2, b2):
    f32 = jnp.float32
    srcr = edge_index[0].reshape(NCHUNKS, CHUNK)
    dstr = edge_index[1].reshape(NCHUNKS, CHUNK)

    ones_deg = jnp.ones((CHUNK, DEG_W), f32)
    zeros_deg = jnp.zeros((RPS, DEG_W), f32)
    zeros_f = jnp.zeros((RPS, H_F), f32)

    w1t = W1.T
    b1r = b1.reshape(1, H_F)
    w2b = W2.T.reshape(D_POLY + 1, H_F, H_F)
    coeff = jnp.asarray(_COEFF, f32)
    g = jnp.tensordot(coeff, w2b, axes=((0,), (0,)))
    g = g.reshape((D_POLY + 1) * H_F, H_F)
    b2r = b2.reshape(1, H_F)

    sc_degree, sc_segsum = _sc_kernels()
    degp = sc_degree(dstr, ones_deg, zeros_deg)
    h, z0, dinv = _tc_lin1(features, degp, w1t, b1r)
    agg1 = sc_segsum(z0, srcr, dstr, zeros_f)
    f1, z1 = _tc_update(h, agg1, dinv)
    agg2 = sc_segsum(z1, srcr, dstr, zeros_f)
    return _tc_final(h, f1, agg2, dinv, g, b2r)


# speedup vs baseline: 1.2252x; 1.0342x over previous
import functools
import math

import jax
import jax.numpy as jnp
import numpy as np
from jax import lax
from jax.experimental import pallas as pl
from jax.experimental.pallas import tpu as pltpu
from jax.experimental.pallas import tpu_sc as plsc

N_NODES = 50000
N_EDGES = 1600000
IN_F = 128
H_F = 32
DEG_W = 16
D_POLY = 2

NC, NS = 2, 16
NW = NC * NS
CHUNK = 125
NCHUNKS = N_EDGES // CHUNK
CPW = NCHUNKS // NW
NPAD = 50048
RPS = NPAD // NS
GRP = 16


def _theta_coeffs(d):
    thetas = []
    for i in range(d + 1):
        p1 = np.zeros(i + 1)
        p1[i] = 0.5 ** i
        m = d - i
        p2 = np.array([math.comb(m, k) * (-0.5) ** k for k in range(m + 1)])
        poly = np.convolve(p1, p2)
        beta = math.gamma(i + 1) * math.gamma(d + 1 - i) / math.gamma(d + 2)
        thetas.append(poly / beta)
    return np.stack(thetas)


_COEFF = _theta_coeffs(D_POLY)


@functools.lru_cache(maxsize=None)
def _sc_kernels():
    mesh = plsc.VectorSubcoreMesh(
        core_axis_name="c", subcore_axis_name="s",
        num_cores=NC, num_subcores=NS)

    params = pltpu.CompilerParams(use_tc_tiling_on_sc=False)

    @functools.partial(
        pl.kernel,
        mesh=mesh,
        compiler_params=params,
        out_type=jax.ShapeDtypeStruct((NC, NPAD, DEG_W), jnp.float32),
        scratch_types=[
            pltpu.VMEM((GRP, CHUNK), jnp.int32),
            pltpu.VMEM((CHUNK, DEG_W), jnp.float32),
            pltpu.VMEM_SHARED((NPAD, DEG_W), jnp.float32),
            pltpu.SemaphoreType.DMA,
        ],
    )
    def _sc_degree(edges_hbm, ones_hbm, zeros_hbm, out_hbm, idx_d, ones_v,
                   acc, dsem):
        c = lax.axis_index("c")
        s = lax.axis_index("s")
        wid = c * NS + s
        start = wid * CPW
        pltpu.sync_copy(zeros_hbm, acc.at[pl.ds(s * RPS, RPS)])
        pltpu.sync_copy(ones_hbm, ones_v)
        plsc.subcore_barrier()

        def group(g, carry):
            pltpu.sync_copy(edges_hbm.at[1, pl.ds(start + g * GRP, GRP)],
                            idx_d)
            for j in range(GRP):
                pltpu.async_copy(ones_v, acc.at[idx_d.at[j]], dsem, add=True)
            for j in range(GRP):
                pltpu.make_async_copy(ones_v, acc.at[idx_d.at[j]],
                                      dsem).wait()
            return carry

        lax.fori_loop(0, CPW // GRP, group, 0)
        plsc.subcore_barrier()
        pltpu.sync_copy(acc.at[pl.ds(s * RPS, RPS)],
                        out_hbm.at[c, pl.ds(s * RPS, RPS)])

    @functools.partial(
        pl.kernel,
        mesh=mesh,
        compiler_params=params,
        out_type=jax.ShapeDtypeStruct((NC, NPAD, H_F), jnp.float32),
        scratch_types=[
            pltpu.VMEM((GRP, CHUNK), jnp.int32),
            pltpu.VMEM((GRP, CHUNK), jnp.int32),
            [pltpu.VMEM((CHUNK, H_F), jnp.float32)] * 4,
            pltpu.VMEM_SHARED((NPAD, H_F), jnp.float32),
            [pltpu.SemaphoreType.DMA] * 4,
            [pltpu.SemaphoreType.DMA] * 4,
        ],
    )
    def _sc_segsum(z_hbm, edges_hbm, zeros_hbm, out_hbm,
                   idx_s, idx_d, rows, acc, gsem, ssem):
        c = lax.axis_index("c")
        s = lax.axis_index("s")
        wid = c * NS + s
        start = wid * CPW
        pltpu.sync_copy(zeros_hbm, acc.at[pl.ds(s * RPS, RPS)])
        plsc.subcore_barrier()

        def group(g, carry):
            pltpu.sync_copy(edges_hbm.at[0, pl.ds(start + g * GRP, GRP)],
                            idx_s)
            pltpu.sync_copy(edges_hbm.at[1, pl.ds(start + g * GRP, GRP)],
                            idx_d)
            pltpu.async_copy(z_hbm.at[idx_s.at[0]], rows[0], gsem[0])
            pltpu.async_copy(z_hbm.at[idx_s.at[1]], rows[1], gsem[1])
            for j in range(GRP):
                b = j % 4
                if j + 2 < GRP:
                    b2 = (j + 2) % 4
                    if j >= 2:
                        pltpu.make_async_copy(
                            rows[b2], acc.at[idx_d.at[j - 2]],
                            ssem[b2]).wait()
                    pltpu.async_copy(z_hbm.at[idx_s.at[j + 2]],
                                     rows[b2], gsem[b2])
                pltpu.make_async_copy(z_hbm.at[idx_s.at[j]],
                                      rows[b], gsem[b]).wait()
                pltpu.async_copy(rows[b], acc.at[idx_d.at[j]],
                                 ssem[b], add=True)
            for j in range(GRP - 4, GRP):
                b = j % 4
                pltpu.make_async_copy(rows[b], acc.at[idx_d.at[j]],
                                      ssem[b]).wait()
            return carry

        lax.fori_loop(0, CPW // GRP, group, 0)
        plsc.subcore_barrier()
        pltpu.sync_copy(acc.at[pl.ds(s * RPS, RPS)],
                        out_hbm.at[c, pl.ds(s * RPS, RPS)])

    return _sc_degree, _sc_segsum


_ROWS_BLK = 2000
_N_BLKS = N_NODES // _ROWS_BLK


def _lin1_body(x_ref, degp_ref, w1t_ref, b1_ref, h_ref, z0_ref, dinv_ref):
    x = x_ref[...]
    h = jnp.dot(x, w1t_ref[...], preferred_element_type=jnp.float32,
                precision=lax.Precision.HIGHEST)
    h = jnp.maximum(h + b1_ref[...], 0.0)
    deg = degp_ref[0, :, 0:1] + degp_ref[1, :, 0:1]
    dinv = lax.rsqrt(jnp.maximum(deg, 1.0))
    h_ref[...] = h
    z0_ref[...] = h * dinv
    dinv_ref[...] = dinv


def _tc_lin1(x, degp, w1t, b1r):
    f32 = jnp.float32
    return pl.pallas_call(
        _lin1_body,
        grid=(_N_BLKS,),
        in_specs=[
            pl.BlockSpec((_ROWS_BLK, IN_F), lambda i: (i, 0)),
            pl.BlockSpec((NC, _ROWS_BLK, DEG_W), lambda i: (0, i, 0)),
            pl.BlockSpec((IN_F, H_F), lambda i: (0, 0)),
            pl.BlockSpec((1, H_F), lambda i: (0, 0)),
        ],
        out_specs=[
            pl.BlockSpec((_ROWS_BLK, H_F), lambda i: (i, 0)),
            pl.BlockSpec((_ROWS_BLK, H_F), lambda i: (i, 0)),
            pl.BlockSpec((_ROWS_BLK, 1), lambda i: (i, 0)),
        ],
        out_shape=[
            jax.ShapeDtypeStruct((N_NODES, H_F), f32),
            jax.ShapeDtypeStruct((N_NODES, H_F), f32),
            jax.ShapeDtypeStruct((N_NODES, 1), f32),
        ],
    )(x, degp, w1t, b1r)


def _update_body(h_ref, aggp_ref, dinv_ref, f1_ref, z1_ref):
    agg = aggp_ref[0] + aggp_ref[1]
    dinv = dinv_ref[...]
    f1 = h_ref[...] - agg * dinv
    f1_ref[...] = f1
    z1_ref[...] = f1 * dinv


def _tc_update(h, aggp, dinv):
    f32 = jnp.float32
    return pl.pallas_call(
        _update_body,
        grid=(_N_BLKS,),
        in_specs=[
            pl.BlockSpec((_ROWS_BLK, H_F), lambda i: (i, 0)),
            pl.BlockSpec((NC, _ROWS_BLK, H_F), lambda i: (0, i, 0)),
            pl.BlockSpec((_ROWS_BLK, 1), lambda i: (i, 0)),
        ],
        out_specs=[
            pl.BlockSpec((_ROWS_BLK, H_F), lambda i: (i, 0)),
            pl.BlockSpec((_ROWS_BLK, H_F), lambda i: (i, 0)),
        ],
        out_shape=[
            jax.ShapeDtypeStruct((N_NODES, H_F), f32),
            jax.ShapeDtypeStruct((N_NODES, H_F), f32),
        ],
    )(h, aggp, dinv)


def _final_body(h_ref, f1_ref, aggp_ref, dinv_ref, g_ref, b2_ref, out_ref):
    agg = aggp_ref[0] + aggp_ref[1]
    f1 = f1_ref[...]
    f2 = f1 - agg * dinv_ref[...]
    cat = jnp.concatenate([h_ref[...], f1, f2], axis=-1)
    out_ref[...] = (
        jnp.dot(cat, g_ref[...], preferred_element_type=jnp.float32,
                precision=lax.Precision.HIGHEST)
        + b2_ref[...]
    )


def _tc_final(h, f1, aggp, dinv, g, b2r):
    return pl.pallas_call(
        _final_body,
        grid=(_N_BLKS,),
        in_specs=[
            pl.BlockSpec((_ROWS_BLK, H_F), lambda i: (i, 0)),
            pl.BlockSpec((_ROWS_BLK, H_F), lambda i: (i, 0)),
            pl.BlockSpec((NC, _ROWS_BLK, H_F), lambda i: (0, i, 0)),
            pl.BlockSpec((_ROWS_BLK, 1), lambda i: (i, 0)),
            pl.BlockSpec((3 * H_F, H_F), lambda i: (0, 0)),
            pl.BlockSpec((1, H_F), lambda i: (0, 0)),
        ],
        out_specs=pl.BlockSpec((_ROWS_BLK, H_F), lambda i: (i, 0)),
        out_shape=jax.ShapeDtypeStruct((N_NODES, H_F), jnp.float32),
    )(h, f1, aggp, dinv, g, b2r)


def kernel(features, edge_index, W1, b1, W2, b2):
    f32 = jnp.float32
    edges = edge_index.reshape(2, NCHUNKS, CHUNK)

    ones_deg = jnp.ones((CHUNK, DEG_W), f32)
    zeros_deg = jnp.zeros((RPS, DEG_W), f32)
    zeros_f = jnp.zeros((RPS, H_F), f32)

    w1t = W1.T
    b1r = b1.reshape(1, H_F)
    w2b = W2.T.reshape(D_POLY + 1, H_F, H_F)
    coeff = jnp.asarray(_COEFF, f32)
    g = jnp.tensordot(coeff, w2b, axes=((0,), (0,)))
    g = g.reshape((D_POLY + 1) * H_F, H_F)
    b2r = b2.reshape(1, H_F)

    sc_degree, sc_segsum = _sc_kernels()
    degp = sc_degree(edges, ones_deg, zeros_deg)
    h, z0, dinv = _tc_lin1(features, degp, w1t, b1r)
    agg1 = sc_segsum(z0, edges, zeros_f)
    f1, z1 = _tc_update(h, agg1, dinv)
    agg2 = sc_segsum(z1, edges, zeros_f)
    return _tc_final(h, f1, agg2, dinv, g, b2r)


# speedup vs baseline: 1.2339x; 1.0071x over previous
import functools
import math

import jax
import jax.numpy as jnp
import numpy as np
from jax import lax
from jax.experimental import pallas as pl
from jax.experimental.pallas import tpu as pltpu
from jax.experimental.pallas import tpu_sc as plsc

N_NODES = 50000
N_EDGES = 1600000
IN_F = 128
H_F = 32
DEG_W = 16
D_POLY = 2

NC, NS = 2, 16
NW = NC * NS
CHUNK = 125
NCHUNKS = N_EDGES // CHUNK
CPW = NCHUNKS // NW
NPAD = 50048
RPS = NPAD // NS
GRP = 16


def _theta_coeffs(d):
    thetas = []
    for i in range(d + 1):
        p1 = np.zeros(i + 1)
        p1[i] = 0.5 ** i
        m = d - i
        p2 = np.array([math.comb(m, k) * (-0.5) ** k for k in range(m + 1)])
        poly = np.convolve(p1, p2)
        beta = math.gamma(i + 1) * math.gamma(d + 1 - i) / math.gamma(d + 2)
        thetas.append(poly / beta)
    return np.stack(thetas)


_COEFF = _theta_coeffs(D_POLY)


@functools.lru_cache(maxsize=None)
def _sc_kernels():
    mesh = plsc.VectorSubcoreMesh(
        core_axis_name="c", subcore_axis_name="s",
        num_cores=NC, num_subcores=NS)

    params = pltpu.CompilerParams(use_tc_tiling_on_sc=False)

    @functools.partial(
        pl.kernel,
        mesh=mesh,
        compiler_params=params,
        out_type=jax.ShapeDtypeStruct((NC, NPAD, DEG_W), jnp.float32),
        scratch_types=[
            pltpu.VMEM((GRP, CHUNK), jnp.int32),
            pltpu.VMEM((CHUNK, DEG_W), jnp.float32),
            pltpu.VMEM_SHARED((NPAD, DEG_W), jnp.float32),
            pltpu.SemaphoreType.DMA,
        ],
    )
    def _sc_degree(edges_hbm, ones_hbm, zeros_hbm, out_hbm, idx_d, ones_v,
                   acc, dsem):
        c = lax.axis_index("c")
        s = lax.axis_index("s")
        wid = c * NS + s
        start = wid * CPW
        pltpu.sync_copy(zeros_hbm, acc.at[pl.ds(s * RPS, RPS)])
        pltpu.sync_copy(ones_hbm, ones_v)
        plsc.subcore_barrier()

        def group(g, carry):
            pltpu.sync_copy(edges_hbm.at[1, pl.ds(start + g * GRP, GRP)],
                            idx_d)
            for j in range(GRP):
                pltpu.async_copy(ones_v, acc.at[idx_d.at[j]], dsem, add=True)
            for j in range(GRP):
                pltpu.make_async_copy(ones_v, acc.at[idx_d.at[j]],
                                      dsem).wait()
            return carry

        lax.fori_loop(0, CPW // GRP, group, 0)
        plsc.subcore_barrier()
        pltpu.sync_copy(acc.at[pl.ds(s * RPS, RPS)],
                        out_hbm.at[c, pl.ds(s * RPS, RPS)])

    @functools.partial(
        pl.kernel,
        mesh=mesh,
        compiler_params=params,
        out_type=jax.ShapeDtypeStruct((NC, NPAD, H_F), jnp.float32),
        scratch_types=[
            pltpu.VMEM((GRP, CHUNK), jnp.int32),
            pltpu.VMEM((GRP, CHUNK), jnp.int32),
            [pltpu.VMEM((CHUNK, H_F), jnp.float32)] * 4,
            pltpu.VMEM_SHARED((NPAD, H_F), jnp.float32),
            [pltpu.SemaphoreType.DMA] * 4,
            [pltpu.SemaphoreType.DMA] * 4,
        ],
    )
    def _sc_segsum(z_hbm, edges_hbm, zeros_hbm, out_hbm,
                   idx_s, idx_d, rows, acc, gsem, ssem):
        c = lax.axis_index("c")
        s = lax.axis_index("s")
        wid = c * NS + s
        start = wid * CPW
        pltpu.sync_copy(zeros_hbm, acc.at[pl.ds(s * RPS, RPS)])
        plsc.subcore_barrier()

        def group(g, carry):
            pltpu.sync_copy(edges_hbm.at[0, pl.ds(start + g * GRP, GRP)],
                            idx_s)
            pltpu.sync_copy(edges_hbm.at[1, pl.ds(start + g * GRP, GRP)],
                            idx_d)
            pltpu.async_copy(z_hbm.at[idx_s.at[0]], rows[0], gsem[0])
            pltpu.async_copy(z_hbm.at[idx_s.at[1]], rows[1], gsem[1])
            for j in range(GRP):
                b = j % 4
                if j + 2 < GRP:
                    b2 = (j + 2) % 4
                    if j >= 2:
                        pltpu.make_async_copy(
                            rows[b2], acc.at[idx_d.at[j - 2]],
                            ssem[b2]).wait()
                    pltpu.async_copy(z_hbm.at[idx_s.at[j + 2]],
                                     rows[b2], gsem[b2])
                pltpu.make_async_copy(z_hbm.at[idx_s.at[j]],
                                      rows[b], gsem[b]).wait()
                pltpu.async_copy(rows[b], acc.at[idx_d.at[j]],
                                 ssem[b], add=True)
            for j in range(GRP - 4, GRP):
                b = j % 4
                pltpu.make_async_copy(rows[b], acc.at[idx_d.at[j]],
                                      ssem[b]).wait()
            return carry

        lax.fori_loop(0, CPW // GRP, group, 0)
        plsc.subcore_barrier()
        pltpu.sync_copy(acc.at[pl.ds(s * RPS, RPS)],
                        out_hbm.at[c, pl.ds(s * RPS, RPS)])

    return _sc_degree, _sc_segsum


_ROWS_BLK = 2000
_N_BLKS = N_NODES // _ROWS_BLK


def _matmul_body(x_ref, w1t_ref, b1_ref, h_ref):
    h = jnp.dot(x_ref[...], w1t_ref[...], preferred_element_type=jnp.float32,
                precision=lax.Precision.HIGHEST)
    h_ref[...] = jnp.maximum(h + b1_ref[...], 0.0)


def _tc_matmul(x, w1t, b1r):
    return pl.pallas_call(
        _matmul_body,
        grid=(_N_BLKS,),
        in_specs=[
            pl.BlockSpec((_ROWS_BLK, IN_F), lambda i: (i, 0)),
            pl.BlockSpec((IN_F, H_F), lambda i: (0, 0)),
            pl.BlockSpec((1, H_F), lambda i: (0, 0)),
        ],
        out_specs=pl.BlockSpec((_ROWS_BLK, H_F), lambda i: (i, 0)),
        out_shape=jax.ShapeDtypeStruct((N_NODES, H_F), jnp.float32),
    )(x, w1t, b1r)


def _scale_body(h_ref, degp_ref, z0_ref, dinv_ref):
    deg = degp_ref[0, :, 0:1] + degp_ref[1, :, 0:1]
    dinv = lax.rsqrt(jnp.maximum(deg, 1.0))
    z0_ref[...] = h_ref[...] * dinv
    dinv_ref[...] = dinv


def _tc_scale(h, degp):
    f32 = jnp.float32
    return pl.pallas_call(
        _scale_body,
        grid=(_N_BLKS,),
        in_specs=[
            pl.BlockSpec((_ROWS_BLK, H_F), lambda i: (i, 0)),
            pl.BlockSpec((NC, _ROWS_BLK, DEG_W), lambda i: (0, i, 0)),
        ],
        out_specs=[
            pl.BlockSpec((_ROWS_BLK, H_F), lambda i: (i, 0)),
            pl.BlockSpec((_ROWS_BLK, 1), lambda i: (i, 0)),
        ],
        out_shape=[
            jax.ShapeDtypeStruct((N_NODES, H_F), f32),
            jax.ShapeDtypeStruct((N_NODES, 1), f32),
        ],
    )(h, degp)


def _update_body(h_ref, aggp_ref, dinv_ref, f1_ref, z1_ref):
    agg = aggp_ref[0] + aggp_ref[1]
    dinv = dinv_ref[...]
    f1 = h_ref[...] - agg * dinv
    f1_ref[...] = f1
    z1_ref[...] = f1 * dinv


def _tc_update(h, aggp, dinv):
    f32 = jnp.float32
    return pl.pallas_call(
        _update_body,
        grid=(_N_BLKS,),
        in_specs=[
            pl.BlockSpec((_ROWS_BLK, H_F), lambda i: (i, 0)),
            pl.BlockSpec((NC, _ROWS_BLK, H_F), lambda i: (0, i, 0)),
            pl.BlockSpec((_ROWS_BLK, 1), lambda i: (i, 0)),
        ],
        out_specs=[
            pl.BlockSpec((_ROWS_BLK, H_F), lambda i: (i, 0)),
            pl.BlockSpec((_ROWS_BLK, H_F), lambda i: (i, 0)),
        ],
        out_shape=[
            jax.ShapeDtypeStruct((N_NODES, H_F), f32),
            jax.ShapeDtypeStruct((N_NODES, H_F), f32),
        ],
    )(h, aggp, dinv)


def _final_body(h_ref, f1_ref, aggp_ref, dinv_ref, g_ref, b2_ref, out_ref):
    agg = aggp_ref[0] + aggp_ref[1]
    f1 = f1_ref[...]
    f2 = f1 - agg * dinv_ref[...]
    cat = jnp.concatenate([h_ref[...], f1, f2], axis=-1)
    out_ref[...] = (
        jnp.dot(cat, g_ref[...], preferred_element_type=jnp.float32,
                precision=lax.Precision.HIGHEST)
        + b2_ref[...]
    )


def _tc_final(h, f1, aggp, dinv, g, b2r):
    return pl.pallas_call(
        _final_body,
        grid=(_N_BLKS,),
        in_specs=[
            pl.BlockSpec((_ROWS_BLK, H_F), lambda i: (i, 0)),
            pl.BlockSpec((_ROWS_BLK, H_F), lambda i: (i, 0)),
            pl.BlockSpec((NC, _ROWS_BLK, H_F), lambda i: (0, i, 0)),
            pl.BlockSpec((_ROWS_BLK, 1), lambda i: (i, 0)),
            pl.BlockSpec((3 * H_F, H_F), lambda i: (0, 0)),
            pl.BlockSpec((1, H_F), lambda i: (0, 0)),
        ],
        out_specs=pl.BlockSpec((_ROWS_BLK, H_F), lambda i: (i, 0)),
        out_shape=jax.ShapeDtypeStruct((N_NODES, H_F), jnp.float32),
    )(h, f1, aggp, dinv, g, b2r)


def kernel(features, edge_index, W1, b1, W2, b2):
    f32 = jnp.float32
    edges = edge_index.reshape(2, NCHUNKS, CHUNK)

    ones_deg = jnp.ones((CHUNK, DEG_W), f32)
    zeros_deg = jnp.zeros((RPS, DEG_W), f32)
    zeros_f = jnp.zeros((RPS, H_F), f32)

    w1t = W1.T
    b1r = b1.reshape(1, H_F)
    w2b = W2.T.reshape(D_POLY + 1, H_F, H_F)
    coeff = jnp.asarray(_COEFF, f32)
    g = jnp.tensordot(coeff, w2b, axes=((0,), (0,)))
    g = g.reshape((D_POLY + 1) * H_F, H_F)
    b2r = b2.reshape(1, H_F)

    sc_degree, sc_segsum = _sc_kernels()
    degp = sc_degree(edges, ones_deg, zeros_deg)
    h = _tc_matmul(features, w1t, b1r)
    z0, dinv = _tc_scale(h, degp)
    agg1 = sc_segsum(z0, edges, zeros_f)
    f1, z1 = _tc_update(h, agg1, dinv)
    agg2 = sc_segsum(z1, edges, zeros_f)
    return _tc_final(h, f1, agg2, dinv, g, b2r)


# speedup vs baseline: 1.2901x; 1.0456x over previous
import functools
import math

import jax
import jax.numpy as jnp
import numpy as np
from jax import lax
from jax.experimental import pallas as pl
from jax.experimental.pallas import tpu as pltpu
from jax.experimental.pallas import tpu_sc as plsc

N_NODES = 50000
N_EDGES = 1600000
IN_F = 128
H_F = 32
DEG_W = 16
D_POLY = 2

NC, NS = 2, 16
NW = NC * NS
CHUNK = 125
NCHUNKS = N_EDGES // CHUNK
CPW = NCHUNKS // NW
NPAD = 50048
RPS = NPAD // NS
GRP = 16


def _theta_coeffs(d):
    thetas = []
    for i in range(d + 1):
        p1 = np.zeros(i + 1)
        p1[i] = 0.5 ** i
        m = d - i
        p2 = np.array([math.comb(m, k) * (-0.5) ** k for k in range(m + 1)])
        poly = np.convolve(p1, p2)
        beta = math.gamma(i + 1) * math.gamma(d + 1 - i) / math.gamma(d + 2)
        thetas.append(poly / beta)
    return np.stack(thetas)


_COEFF = _theta_coeffs(D_POLY)


@functools.lru_cache(maxsize=None)
def _sc_kernels():
    mesh = plsc.VectorSubcoreMesh(
        core_axis_name="c", subcore_axis_name="s",
        num_cores=NC, num_subcores=NS)

    params = pltpu.CompilerParams(use_tc_tiling_on_sc=False)

    @functools.partial(
        pl.kernel,
        mesh=mesh,
        compiler_params=params,
        out_type=jax.ShapeDtypeStruct((NC, NPAD, DEG_W), jnp.float32),
        scratch_types=[
            pltpu.VMEM((GRP, CHUNK), jnp.int32),
            pltpu.VMEM((CHUNK, DEG_W), jnp.float32),
            pltpu.VMEM_SHARED((NPAD, DEG_W), jnp.float32),
            pltpu.SemaphoreType.DMA,
        ],
    )
    def _sc_degree(edges_hbm, ones_hbm, zeros_hbm, out_hbm, idx_d, ones_v,
                   acc, dsem):
        c = lax.axis_index("c")
        s = lax.axis_index("s")
        wid = c * NS + s
        start = wid * CPW
        pltpu.sync_copy(zeros_hbm, acc.at[pl.ds(s * RPS, RPS)])
        pltpu.sync_copy(ones_hbm, ones_v)
        plsc.subcore_barrier()

        def group(g, carry):
            pltpu.sync_copy(edges_hbm.at[1, pl.ds(start + g * GRP, GRP)],
                            idx_d)
            for j in range(GRP):
                pltpu.async_copy(ones_v, acc.at[idx_d.at[j]], dsem, add=True)
            for j in range(GRP):
                pltpu.make_async_copy(ones_v, acc.at[idx_d.at[j]],
                                      dsem).wait()
            return carry

        lax.fori_loop(0, CPW // GRP, group, 0)
        plsc.subcore_barrier()
        pltpu.sync_copy(acc.at[pl.ds(s * RPS, RPS)],
                        out_hbm.at[c, pl.ds(s * RPS, RPS)])

    @functools.partial(
        pl.kernel,
        mesh=mesh,
        compiler_params=params,
        out_type=jax.ShapeDtypeStruct((NC, NPAD, H_F), jnp.float32),
        scratch_types=[
            pltpu.VMEM((GRP, CHUNK), jnp.int32),
            pltpu.VMEM((GRP, CHUNK), jnp.int32),
            [pltpu.VMEM((CHUNK, H_F), jnp.float32)] * 4,
            pltpu.VMEM_SHARED((NPAD, H_F), jnp.float32),
            [pltpu.SemaphoreType.DMA] * 4,
            [pltpu.SemaphoreType.DMA] * 4,
        ],
    )
    def _sc_segsum(z_hbm, edges_hbm, zeros_hbm, out_hbm,
                   idx_s, idx_d, rows, acc, gsem, ssem):
        c = lax.axis_index("c")
        s = lax.axis_index("s")
        wid = c * NS + s
        start = wid * CPW
        pltpu.sync_copy(zeros_hbm, acc.at[pl.ds(s * RPS, RPS)])
        plsc.subcore_barrier()

        def group(g, carry):
            pltpu.sync_copy(edges_hbm.at[0, pl.ds(start + g * GRP, GRP)],
                            idx_s)
            pltpu.sync_copy(edges_hbm.at[1, pl.ds(start + g * GRP, GRP)],
                            idx_d)
            pltpu.async_copy(z_hbm.at[idx_s.at[0]], rows[0], gsem[0])
            pltpu.async_copy(z_hbm.at[idx_s.at[1]], rows[1], gsem[1])
            for j in range(GRP):
                b = j % 4
                if j + 2 < GRP:
                    b2 = (j + 2) % 4
                    if j >= 2:
                        pltpu.make_async_copy(
                            rows[b2], acc.at[idx_d.at[j - 2]],
                            ssem[b2]).wait()
                    pltpu.async_copy(z_hbm.at[idx_s.at[j + 2]],
                                     rows[b2], gsem[b2])
                pltpu.make_async_copy(z_hbm.at[idx_s.at[j]],
                                      rows[b], gsem[b]).wait()
                pltpu.async_copy(rows[b], acc.at[idx_d.at[j]],
                                 ssem[b], add=True)
            for j in range(GRP - 4, GRP):
                b = j % 4
                pltpu.make_async_copy(rows[b], acc.at[idx_d.at[j]],
                                      ssem[b]).wait()
            return carry

        lax.fori_loop(0, CPW // GRP, group, 0)
        plsc.subcore_barrier()
        pltpu.sync_copy(acc.at[pl.ds(s * RPS, RPS)],
                        out_hbm.at[c, pl.ds(s * RPS, RPS)])

    return _sc_degree, _sc_segsum


_ROWS_BLK = 2048
_N_BLKS = 25
_ZROWS = _ROWS_BLK // 4


def _to_flat128(z):
    r = z.reshape(_ZROWS, 4, H_F)
    return jnp.concatenate(
        [r[:, 0, :], r[:, 1, :], r[:, 2, :], r[:, 3, :]], axis=-1)


def _matmul_body(x_ref, w1t_ref, b1_ref, h_ref):
    h = jnp.dot(x_ref[...], w1t_ref[...], preferred_element_type=jnp.float32,
                precision=lax.Precision.HIGHEST)
    h_ref[...] = jnp.maximum(h + b1_ref[...], 0.0)


def _tc_matmul(x, w1t, b1r):
    return pl.pallas_call(
        _matmul_body,
        grid=(_N_BLKS,),
        in_specs=[
            pl.BlockSpec((_ROWS_BLK, IN_F), lambda i: (i, 0)),
            pl.BlockSpec((IN_F, H_F), lambda i: (0, 0)),
            pl.BlockSpec((1, H_F), lambda i: (0, 0)),
        ],
        out_specs=pl.BlockSpec((_ROWS_BLK, H_F), lambda i: (i, 0)),
        out_shape=jax.ShapeDtypeStruct((N_NODES, H_F), jnp.float32),
    )(x, w1t, b1r)


def _scale_body(h_ref, degp_ref, z0_ref, dinv_ref):
    deg = degp_ref[0, :, 0:1] + degp_ref[1, :, 0:1]
    dinv = lax.rsqrt(jnp.maximum(deg, 1.0))
    z0_ref[...] = _to_flat128(h_ref[...] * dinv)
    dinv_ref[...] = dinv


def _tc_scale(h, degp):
    f32 = jnp.float32
    return pl.pallas_call(
        _scale_body,
        grid=(_N_BLKS,),
        in_specs=[
            pl.BlockSpec((_ROWS_BLK, H_F), lambda i: (i, 0)),
            pl.BlockSpec((NC, _ROWS_BLK, DEG_W), lambda i: (0, i, 0)),
        ],
        out_specs=[
            pl.BlockSpec((_ZROWS, 4 * H_F), lambda i: (i, 0)),
            pl.BlockSpec((_ROWS_BLK, 1), lambda i: (i, 0)),
        ],
        out_shape=[
            jax.ShapeDtypeStruct((_N_BLKS * _ZROWS, 4 * H_F), f32),
            jax.ShapeDtypeStruct((N_NODES, 1), f32),
        ],
    )(h, degp)


def _update_body(h_ref, aggp_ref, dinv_ref, f1_ref, z1_ref):
    agg = aggp_ref[0] + aggp_ref[1]
    dinv = dinv_ref[...]
    f1 = h_ref[...] - agg * dinv
    f1_ref[...] = f1
    z1_ref[...] = _to_flat128(f1 * dinv)


def _tc_update(h, aggp, dinv):
    f32 = jnp.float32
    return pl.pallas_call(
        _update_body,
        grid=(_N_BLKS,),
        in_specs=[
            pl.BlockSpec((_ROWS_BLK, H_F), lambda i: (i, 0)),
            pl.BlockSpec((NC, _ROWS_BLK, H_F), lambda i: (0, i, 0)),
            pl.BlockSpec((_ROWS_BLK, 1), lambda i: (i, 0)),
        ],
        out_specs=[
            pl.BlockSpec((_ROWS_BLK, H_F), lambda i: (i, 0)),
            pl.BlockSpec((_ZROWS, 4 * H_F), lambda i: (i, 0)),
        ],
        out_shape=[
            jax.ShapeDtypeStruct((N_NODES, H_F), f32),
            jax.ShapeDtypeStruct((_N_BLKS * _ZROWS, 4 * H_F), f32),
        ],
    )(h, aggp, dinv)


def _final_body(h_ref, f1_ref, aggp_ref, dinv_ref, g_ref, b2_ref, out_ref):
    agg = aggp_ref[0] + aggp_ref[1]
    f1 = f1_ref[...]
    f2 = f1 - agg * dinv_ref[...]
    cat = jnp.concatenate([h_ref[...], f1, f2], axis=-1)
    out_ref[...] = (
        jnp.dot(cat, g_ref[...], preferred_element_type=jnp.float32,
                precision=lax.Precision.HIGHEST)
        + b2_ref[...]
    )


def _tc_final(h, f1, aggp, dinv, g, b2r):
    return pl.pallas_call(
        _final_body,
        grid=(_N_BLKS,),
        in_specs=[
            pl.BlockSpec((_ROWS_BLK, H_F), lambda i: (i, 0)),
            pl.BlockSpec((_ROWS_BLK, H_F), lambda i: (i, 0)),
            pl.BlockSpec((NC, _ROWS_BLK, H_F), lambda i: (0, i, 0)),
            pl.BlockSpec((_ROWS_BLK, 1), lambda i: (i, 0)),
            pl.BlockSpec((3 * H_F, H_F), lambda i: (0, 0)),
            pl.BlockSpec((1, H_F), lambda i: (0, 0)),
        ],
        out_specs=pl.BlockSpec((_ROWS_BLK, H_F), lambda i: (i, 0)),
        out_shape=jax.ShapeDtypeStruct((N_NODES, H_F), jnp.float32),
    )(h, f1, aggp, dinv, g, b2r)


def kernel(features, edge_index, W1, b1, W2, b2):
    f32 = jnp.float32
    edges = edge_index.reshape(2, NCHUNKS, CHUNK)

    ones_deg = jnp.ones((CHUNK, DEG_W), f32)
    zeros_deg = jnp.zeros((RPS, DEG_W), f32)
    zeros_f = jnp.zeros((RPS, H_F), f32)

    w1t = W1.T
    b1r = b1.reshape(1, H_F)
    w2b = W2.T.reshape(D_POLY + 1, H_F, H_F)
    coeff = jnp.asarray(_COEFF, f32)
    g = jnp.tensordot(coeff, w2b, axes=((0,), (0,)))
    g = g.reshape((D_POLY + 1) * H_F, H_F)
    b2r = b2.reshape(1, H_F)

    sc_degree, sc_segsum = _sc_kernels()
    degp = sc_degree(edges, ones_deg, zeros_deg)
    h = _tc_matmul(features, w1t, b1r)
    z0f, dinv = _tc_scale(h, degp)
    agg1 = sc_segsum(z0f.reshape(_N_BLKS * _ROWS_BLK, H_F), edges, zeros_f)
    f1, z1f = _tc_update(h, agg1, dinv)
    agg2 = sc_segsum(z1f.reshape(_N_BLKS * _ROWS_BLK, H_F), edges, zeros_f)
    return _tc_final(h, f1, agg2, dinv, g, b2r)


# speedup vs baseline: 1.3050x; 1.0115x over previous
import functools
import math

import jax
import jax.numpy as jnp
import numpy as np
from jax import lax
from jax.experimental import pallas as pl
from jax.experimental.pallas import tpu as pltpu
from jax.experimental.pallas import tpu_sc as plsc

N_NODES = 50000
N_EDGES = 1600000
IN_F = 128
H_F = 32
DEG_W = 16
D_POLY = 2

NC, NS = 2, 16
NW = NC * NS
CHUNK = 125
NCHUNKS = N_EDGES // CHUNK
CPW = NCHUNKS // NW
NPAD = 51200
RPS = NPAD // NS
GRP = 16


def _theta_coeffs(d):
    thetas = []
    for i in range(d + 1):
        p1 = np.zeros(i + 1)
        p1[i] = 0.5 ** i
        m = d - i
        p2 = np.array([math.comb(m, k) * (-0.5) ** k for k in range(m + 1)])
        poly = np.convolve(p1, p2)
        beta = math.gamma(i + 1) * math.gamma(d + 1 - i) / math.gamma(d + 2)
        thetas.append(poly / beta)
    return np.stack(thetas)


_COEFF = _theta_coeffs(D_POLY)


@functools.lru_cache(maxsize=None)
def _sc_kernels():
    mesh = plsc.VectorSubcoreMesh(
        core_axis_name="c", subcore_axis_name="s",
        num_cores=NC, num_subcores=NS)

    params = pltpu.CompilerParams(use_tc_tiling_on_sc=False)

    @functools.partial(
        pl.kernel,
        mesh=mesh,
        compiler_params=params,
        out_type=jax.ShapeDtypeStruct((NC, NPAD, DEG_W), jnp.float32),
        scratch_types=[
            pltpu.VMEM((GRP, CHUNK), jnp.int32),
            pltpu.VMEM((CHUNK, DEG_W), jnp.float32),
            pltpu.VMEM_SHARED((NPAD, DEG_W), jnp.float32),
            pltpu.SemaphoreType.DMA,
        ],
    )
    def _sc_degree(edges_hbm, ones_hbm, zeros_hbm, out_hbm, idx_d, ones_v,
                   acc, dsem):
        c = lax.axis_index("c")
        s = lax.axis_index("s")
        wid = c * NS + s
        start = wid * CPW
        pltpu.sync_copy(zeros_hbm, acc.at[pl.ds(s * RPS, RPS)])
        pltpu.sync_copy(ones_hbm, ones_v)
        plsc.subcore_barrier()

        def group(g, carry):
            pltpu.sync_copy(edges_hbm.at[1, pl.ds(start + g * GRP, GRP)],
                            idx_d)
            for j in range(GRP):
                pltpu.async_copy(ones_v, acc.at[idx_d.at[j]], dsem, add=True)
            for j in range(GRP):
                pltpu.make_async_copy(ones_v, acc.at[idx_d.at[j]],
                                      dsem).wait()
            return carry

        lax.fori_loop(0, CPW // GRP, group, 0)
        plsc.subcore_barrier()
        pltpu.sync_copy(acc.at[pl.ds(s * RPS, RPS)],
                        out_hbm.at[c, pl.ds(s * RPS, RPS)])

    @functools.partial(
        pl.kernel,
        mesh=mesh,
        compiler_params=params,
        out_type=jax.ShapeDtypeStruct((NC, NPAD, H_F), jnp.float32),
        scratch_types=[
            pltpu.VMEM((GRP, CHUNK), jnp.int32),
            pltpu.VMEM((GRP, CHUNK), jnp.int32),
            [pltpu.VMEM((CHUNK, H_F), jnp.float32)] * 4,
            pltpu.VMEM_SHARED((NPAD, H_F), jnp.float32),
            [pltpu.SemaphoreType.DMA] * 4,
            [pltpu.SemaphoreType.DMA] * 4,
        ],
    )
    def _sc_segsum(z_hbm, edges_hbm, zeros_hbm, out_hbm,
                   idx_s, idx_d, rows, acc, gsem, ssem):
        c = lax.axis_index("c")
        s = lax.axis_index("s")
        wid = c * NS + s
        start = wid * CPW
        pltpu.sync_copy(zeros_hbm, acc.at[pl.ds(s * RPS, RPS)])
        plsc.subcore_barrier()

        def group(g, carry):
            pltpu.sync_copy(edges_hbm.at[0, pl.ds(start + g * GRP, GRP)],
                            idx_s)
            pltpu.sync_copy(edges_hbm.at[1, pl.ds(start + g * GRP, GRP)],
                            idx_d)
            pltpu.async_copy(z_hbm.at[idx_s.at[0]], rows[0], gsem[0])
            pltpu.async_copy(z_hbm.at[idx_s.at[1]], rows[1], gsem[1])
            for j in range(GRP):
                b = j % 4
                if j + 2 < GRP:
                    b2 = (j + 2) % 4
                    if j >= 2:
                        pltpu.make_async_copy(
                            rows[b2], acc.at[idx_d.at[j - 2]],
                            ssem[b2]).wait()
                    pltpu.async_copy(z_hbm.at[idx_s.at[j + 2]],
                                     rows[b2], gsem[b2])
                pltpu.make_async_copy(z_hbm.at[idx_s.at[j]],
                                      rows[b], gsem[b]).wait()
                pltpu.async_copy(rows[b], acc.at[idx_d.at[j]],
                                 ssem[b], add=True)
            for j in range(GRP - 4, GRP):
                b = j % 4
                pltpu.make_async_copy(rows[b], acc.at[idx_d.at[j]],
                                      ssem[b]).wait()
            return carry

        lax.fori_loop(0, CPW // GRP, group, 0)
        plsc.subcore_barrier()
        pltpu.sync_copy(acc.at[pl.ds(s * RPS, RPS)],
                        out_hbm.at[c, pl.ds(s * RPS, RPS)])

    return _sc_degree, _sc_segsum


_ROWS_BLK = 2048
_N_BLKS = 25
_ZROWS = _ROWS_BLK // 4


def _from_flat128(blk):
    parts = [blk[:, c * H_F:(c + 1) * H_F] for c in range(4)]
    return jnp.stack(parts, axis=1).reshape(_ROWS_BLK, H_F)


def _to_flat128(z):
    r = z.reshape(_ZROWS, 4, H_F)
    return jnp.concatenate(
        [r[:, 0, :], r[:, 1, :], r[:, 2, :], r[:, 3, :]], axis=-1)


def _matmul_body(x_ref, w1t_ref, b1_ref, h_ref):
    h = jnp.dot(x_ref[...], w1t_ref[...], preferred_element_type=jnp.float32,
                precision=lax.Precision.HIGHEST)
    h_ref[...] = jnp.maximum(h + b1_ref[...], 0.0)


def _tc_matmul(x, w1t, b1r):
    return pl.pallas_call(
        _matmul_body,
        grid=(_N_BLKS,),
        in_specs=[
            pl.BlockSpec((_ROWS_BLK, IN_F), lambda i: (i, 0)),
            pl.BlockSpec((IN_F, H_F), lambda i: (0, 0)),
            pl.BlockSpec((1, H_F), lambda i: (0, 0)),
        ],
        out_specs=pl.BlockSpec((_ROWS_BLK, H_F), lambda i: (i, 0)),
        out_shape=jax.ShapeDtypeStruct((N_NODES, H_F), jnp.float32),
    )(x, w1t, b1r)


def _scale_body(h_ref, degp_ref, z0_ref, dinv_ref):
    deg = degp_ref[0, :, 0:1] + degp_ref[1, :, 0:1]
    dinv = lax.rsqrt(jnp.maximum(deg, 1.0))
    z0_ref[...] = _to_flat128(h_ref[...] * dinv)
    dinv_ref[...] = dinv


def _tc_scale(h, degp):
    f32 = jnp.float32
    return pl.pallas_call(
        _scale_body,
        grid=(_N_BLKS,),
        in_specs=[
            pl.BlockSpec((_ROWS_BLK, H_F), lambda i: (i, 0)),
            pl.BlockSpec((NC, _ROWS_BLK, DEG_W), lambda i: (0, i, 0)),
        ],
        out_specs=[
            pl.BlockSpec((_ZROWS, 4 * H_F), lambda i: (i, 0)),
            pl.BlockSpec((_ROWS_BLK, 1), lambda i: (i, 0)),
        ],
        out_shape=[
            jax.ShapeDtypeStruct((_N_BLKS * _ZROWS, 4 * H_F), f32),
            jax.ShapeDtypeStruct((N_NODES, 1), f32),
        ],
    )(h, degp)


def _update_body(h_ref, agg0_ref, agg1_ref, dinv_ref, f1_ref, z1_ref):
    agg = _from_flat128(agg0_ref[...] + agg1_ref[...])
    dinv = dinv_ref[...]
    f1 = h_ref[...] - agg * dinv
    f1_ref[...] = f1
    z1_ref[...] = _to_flat128(f1 * dinv)


def _tc_update(h, aggf, dinv):
    f32 = jnp.float32
    return pl.pallas_call(
        _update_body,
        grid=(_N_BLKS,),
        in_specs=[
            pl.BlockSpec((_ROWS_BLK, H_F), lambda i: (i, 0)),
            pl.BlockSpec((_ZROWS, 4 * H_F), lambda i: (i, 0)),
            pl.BlockSpec((_ZROWS, 4 * H_F), lambda i: (_N_BLKS + i, 0)),
            pl.BlockSpec((_ROWS_BLK, 1), lambda i: (i, 0)),
        ],
        out_specs=[
            pl.BlockSpec((_ROWS_BLK, H_F), lambda i: (i, 0)),
            pl.BlockSpec((_ZROWS, 4 * H_F), lambda i: (i, 0)),
        ],
        out_shape=[
            jax.ShapeDtypeStruct((N_NODES, H_F), f32),
            jax.ShapeDtypeStruct((_N_BLKS * _ZROWS, 4 * H_F), f32),
        ],
    )(h, aggf, aggf, dinv)


def _final_body(h_ref, f1_ref, agg0_ref, agg1_ref, dinv_ref, g_ref, b2_ref,
                out_ref):
    agg = _from_flat128(agg0_ref[...] + agg1_ref[...])
    f1 = f1_ref[...]
    f2 = f1 - agg * dinv_ref[...]
    cat = jnp.concatenate([h_ref[...], f1, f2], axis=-1)
    out_ref[...] = (
        jnp.dot(cat, g_ref[...], preferred_element_type=jnp.float32,
                precision=lax.Precision.HIGHEST)
        + b2_ref[...]
    )


def _tc_final(h, f1, aggf, dinv, g, b2r):
    return pl.pallas_call(
        _final_body,
        grid=(_N_BLKS,),
        in_specs=[
            pl.BlockSpec((_ROWS_BLK, H_F), lambda i: (i, 0)),
            pl.BlockSpec((_ROWS_BLK, H_F), lambda i: (i, 0)),
            pl.BlockSpec((_ZROWS, 4 * H_F), lambda i: (i, 0)),
            pl.BlockSpec((_ZROWS, 4 * H_F), lambda i: (_N_BLKS + i, 0)),
            pl.BlockSpec((_ROWS_BLK, 1), lambda i: (i, 0)),
            pl.BlockSpec((3 * H_F, H_F), lambda i: (0, 0)),
            pl.BlockSpec((1, H_F), lambda i: (0, 0)),
        ],
        out_specs=pl.BlockSpec((_ROWS_BLK, H_F), lambda i: (i, 0)),
        out_shape=jax.ShapeDtypeStruct((N_NODES, H_F), jnp.float32),
    )(h, f1, aggf, aggf, dinv, g, b2r)


def kernel(features, edge_index, W1, b1, W2, b2):
    f32 = jnp.float32
    edges = edge_index.reshape(2, NCHUNKS, CHUNK)

    ones_deg = jnp.ones((CHUNK, DEG_W), f32)
    zeros_deg = jnp.zeros((RPS, DEG_W), f32)
    zeros_f = jnp.zeros((RPS, H_F), f32)

    w1t = W1.T
    b1r = b1.reshape(1, H_F)
    w2b = W2.T.reshape(D_POLY + 1, H_F, H_F)
    coeff = jnp.asarray(_COEFF, f32)
    g = jnp.tensordot(coeff, w2b, axes=((0,), (0,)))
    g = g.reshape((D_POLY + 1) * H_F, H_F)
    b2r = b2.reshape(1, H_F)

    sc_degree, sc_segsum = _sc_kernels()
    degp = sc_degree(edges, ones_deg, zeros_deg)
    h = _tc_matmul(features, w1t, b1r)
    z0f, dinv = _tc_scale(h, degp)
    agg1 = sc_segsum(z0f.reshape(_N_BLKS * _ROWS_BLK, H_F), edges, zeros_f)
    agg1f = agg1.reshape(NC * NPAD * H_F // 128, 128)
    f1, z1f = _tc_update(h, agg1f, dinv)
    agg2 = sc_segsum(z1f.reshape(_N_BLKS * _ROWS_BLK, H_F), edges, zeros_f)
    agg2f = agg2.reshape(NC * NPAD * H_F // 128, 128)
    return _tc_final(h, f1, agg2f, dinv, g, b2r)


# speedup vs baseline: 1.3716x; 1.0510x over previous
import functools
import math

import jax
import jax.numpy as jnp
import numpy as np
from jax import lax
from jax.experimental import pallas as pl
from jax.experimental.pallas import tpu as pltpu
from jax.experimental.pallas import tpu_sc as plsc

N_NODES = 50000
N_EDGES = 1600000
IN_F = 128
H_F = 32
DEG_W = 16
D_POLY = 2

NC, NS = 2, 16
NW = NC * NS
CHUNK = 125
NCHUNKS = N_EDGES // CHUNK
CPW = NCHUNKS // NW
NPAD = 51200
RPS = NPAD // NS
GRP = 16


def _theta_coeffs(d):
    thetas = []
    for i in range(d + 1):
        p1 = np.zeros(i + 1)
        p1[i] = 0.5 ** i
        m = d - i
        p2 = np.array([math.comb(m, k) * (-0.5) ** k for k in range(m + 1)])
        poly = np.convolve(p1, p2)
        beta = math.gamma(i + 1) * math.gamma(d + 1 - i) / math.gamma(d + 2)
        thetas.append(poly / beta)
    return np.stack(thetas)


_COEFF = _theta_coeffs(D_POLY)


@functools.lru_cache(maxsize=None)
def _sc_kernels():
    mesh = plsc.VectorSubcoreMesh(
        core_axis_name="c", subcore_axis_name="s",
        num_cores=NC, num_subcores=NS)

    params = pltpu.CompilerParams(use_tc_tiling_on_sc=False)

    @functools.partial(
        pl.kernel,
        mesh=mesh,
        compiler_params=params,
        out_type=jax.ShapeDtypeStruct((NC, NPAD, DEG_W), jnp.float32),
        scratch_types=[
            pltpu.VMEM((GRP, CHUNK), jnp.int32),
            pltpu.VMEM((CHUNK, DEG_W), jnp.float32),
            pltpu.VMEM_SHARED((NPAD, DEG_W), jnp.float32),
            pltpu.SemaphoreType.DMA,
        ],
    )
    def _sc_degree(edges_hbm, ones_hbm, zeros_hbm, out_hbm, idx_d, ones_v,
                   acc, dsem):
        c = lax.axis_index("c")
        s = lax.axis_index("s")
        wid = c * NS + s
        start = wid * CPW
        pltpu.sync_copy(zeros_hbm, acc.at[pl.ds(s * RPS, RPS)])
        pltpu.sync_copy(ones_hbm, ones_v)
        plsc.subcore_barrier()

        def group(g, carry):
            pltpu.sync_copy(edges_hbm.at[1, pl.ds(start + g * GRP, GRP)],
                            idx_d)
            for j in range(GRP):
                pltpu.async_copy(ones_v, acc.at[idx_d.at[j]], dsem, add=True)
            for j in range(GRP):
                pltpu.make_async_copy(ones_v, acc.at[idx_d.at[j]],
                                      dsem).wait()
            return carry

        lax.fori_loop(0, CPW // GRP, group, 0)
        plsc.subcore_barrier()
        pltpu.sync_copy(acc.at[pl.ds(s * RPS, RPS)],
                        out_hbm.at[c, pl.ds(s * RPS, RPS)])

    @functools.partial(
        pl.kernel,
        mesh=mesh,
        compiler_params=params,
        out_type=jax.ShapeDtypeStruct((NC, NPAD, H_F), jnp.float32),
        scratch_types=[
            pltpu.VMEM((GRP, CHUNK), jnp.int32),
            pltpu.VMEM((GRP, CHUNK), jnp.int32),
            [pltpu.VMEM((CHUNK, H_F), jnp.float32)] * 6,
            pltpu.VMEM_SHARED((NPAD, H_F), jnp.float32),
            [pltpu.SemaphoreType.DMA] * 6,
            [pltpu.SemaphoreType.DMA] * 6,
        ],
    )
    def _sc_segsum(z_hbm, edges_hbm, zeros_hbm, out_hbm,
                   idx_s, idx_d, rows, acc, gsem, ssem):
        c = lax.axis_index("c")
        s = lax.axis_index("s")
        wid = c * NS + s
        start = wid * CPW
        pltpu.sync_copy(zeros_hbm, acc.at[pl.ds(s * RPS, RPS)])
        plsc.subcore_barrier()

        def group(g, carry):
            pltpu.sync_copy(edges_hbm.at[0, pl.ds(start + g * GRP, GRP)],
                            idx_s)
            pltpu.sync_copy(edges_hbm.at[1, pl.ds(start + g * GRP, GRP)],
                            idx_d)
            for p in range(3):
                pltpu.async_copy(z_hbm.at[idx_s.at[p]], rows[p], gsem[p])
            for j in range(GRP):
                b = j % 6
                if j + 3 < GRP:
                    b2 = (j + 3) % 6
                    if j >= 3:
                        pltpu.make_async_copy(
                            rows[b2], acc.at[idx_d.at[j - 3]],
                            ssem[b2]).wait()
                    pltpu.async_copy(z_hbm.at[idx_s.at[j + 3]],
                                     rows[b2], gsem[b2])
                pltpu.make_async_copy(z_hbm.at[idx_s.at[j]],
                                      rows[b], gsem[b]).wait()
                pltpu.async_copy(rows[b], acc.at[idx_d.at[j]],
                                 ssem[b], add=True)
            for j in range(GRP - 6, GRP):
                b = j % 6
                pltpu.make_async_copy(rows[b], acc.at[idx_d.at[j]],
                                      ssem[b]).wait()
            return carry

        lax.fori_loop(0, CPW // GRP, group, 0)
        plsc.subcore_barrier()
        pltpu.sync_copy(acc.at[pl.ds(s * RPS, RPS)],
                        out_hbm.at[c, pl.ds(s * RPS, RPS)])

    return _sc_degree, _sc_segsum


_ROWS_BLK = 2048
_N_BLKS = 25
_ZROWS = _ROWS_BLK // 4


def _from_flat128(blk):
    parts = [blk[:, c * H_F:(c + 1) * H_F] for c in range(4)]
    return jnp.stack(parts, axis=1).reshape(_ROWS_BLK, H_F)


def _to_flat128(z):
    r = z.reshape(_ZROWS, 4, H_F)
    return jnp.concatenate(
        [r[:, 0, :], r[:, 1, :], r[:, 2, :], r[:, 3, :]], axis=-1)


def _matmul_body(x_ref, w1t_ref, b1_ref, h_ref):
    h = jnp.dot(x_ref[...], w1t_ref[...], preferred_element_type=jnp.float32,
                precision=lax.Precision.HIGHEST)
    h_ref[...] = jnp.maximum(h + b1_ref[...], 0.0)


def _tc_matmul(x, w1t, b1r):
    return pl.pallas_call(
        _matmul_body,
        grid=(_N_BLKS,),
        in_specs=[
            pl.BlockSpec((_ROWS_BLK, IN_F), lambda i: (i, 0)),
            pl.BlockSpec((IN_F, H_F), lambda i: (0, 0)),
            pl.BlockSpec((1, H_F), lambda i: (0, 0)),
        ],
        out_specs=pl.BlockSpec((_ROWS_BLK, H_F), lambda i: (i, 0)),
        out_shape=jax.ShapeDtypeStruct((N_NODES, H_F), jnp.float32),
    )(x, w1t, b1r)


def _scale_body(h_ref, degp_ref, z0_ref, dinv_ref):
    deg = degp_ref[0, :, 0:1] + degp_ref[1, :, 0:1]
    dinv = lax.rsqrt(jnp.maximum(deg, 1.0))
    z0_ref[...] = _to_flat128(h_ref[...] * dinv)
    dinv_ref[...] = dinv


def _tc_scale(h, degp):
    f32 = jnp.float32
    return pl.pallas_call(
        _scale_body,
        grid=(_N_BLKS,),
        in_specs=[
            pl.BlockSpec((_ROWS_BLK, H_F), lambda i: (i, 0)),
            pl.BlockSpec((NC, _ROWS_BLK, DEG_W), lambda i: (0, i, 0)),
        ],
        out_specs=[
            pl.BlockSpec((_ZROWS, 4 * H_F), lambda i: (i, 0)),
            pl.BlockSpec((_ROWS_BLK, 1), lambda i: (i, 0)),
        ],
        out_shape=[
            jax.ShapeDtypeStruct((_N_BLKS * _ZROWS, 4 * H_F), f32),
            jax.ShapeDtypeStruct((N_NODES, 1), f32),
        ],
    )(h, degp)


def _update_body(h_ref, agg0_ref, agg1_ref, dinv_ref, f1_ref, z1_ref):
    agg = _from_flat128(agg0_ref[...] + agg1_ref[...])
    dinv = dinv_ref[...]
    f1 = h_ref[...] - agg * dinv
    f1_ref[...] = f1
    z1_ref[...] = _to_flat128(f1 * dinv)


def _tc_update(h, aggf, dinv):
    f32 = jnp.float32
    return pl.pallas_call(
        _update_body,
        grid=(_N_BLKS,),
        in_specs=[
            pl.BlockSpec((_ROWS_BLK, H_F), lambda i: (i, 0)),
            pl.BlockSpec((_ZROWS, 4 * H_F), lambda i: (i, 0)),
            pl.BlockSpec((_ZROWS, 4 * H_F), lambda i: (_N_BLKS + i, 0)),
            pl.BlockSpec((_ROWS_BLK, 1), lambda i: (i, 0)),
        ],
        out_specs=[
            pl.BlockSpec((_ROWS_BLK, H_F), lambda i: (i, 0)),
            pl.BlockSpec((_ZROWS, 4 * H_F), lambda i: (i, 0)),
        ],
        out_shape=[
            jax.ShapeDtypeStruct((N_NODES, H_F), f32),
            jax.ShapeDtypeStruct((_N_BLKS * _ZROWS, 4 * H_F), f32),
        ],
    )(h, aggf, aggf, dinv)


def _final_body(h_ref, f1_ref, agg0_ref, agg1_ref, dinv_ref, g_ref, b2_ref,
                out_ref):
    agg = _from_flat128(agg0_ref[...] + agg1_ref[...])
    f1 = f1_ref[...]
    f2 = f1 - agg * dinv_ref[...]
    cat = jnp.concatenate([h_ref[...], f1, f2], axis=-1)
    out_ref[...] = (
        jnp.dot(cat, g_ref[...], preferred_element_type=jnp.float32,
                precision=lax.Precision.HIGHEST)
        + b2_ref[...]
    )


def _tc_final(h, f1, aggf, dinv, g, b2r):
    return pl.pallas_call(
        _final_body,
        grid=(_N_BLKS,),
        in_specs=[
            pl.BlockSpec((_ROWS_BLK, H_F), lambda i: (i, 0)),
            pl.BlockSpec((_ROWS_BLK, H_F), lambda i: (i, 0)),
            pl.BlockSpec((_ZROWS, 4 * H_F), lambda i: (i, 0)),
            pl.BlockSpec((_ZROWS, 4 * H_F), lambda i: (_N_BLKS + i, 0)),
            pl.BlockSpec((_ROWS_BLK, 1), lambda i: (i, 0)),
            pl.BlockSpec((3 * H_F, H_F), lambda i: (0, 0)),
            pl.BlockSpec((1, H_F), lambda i: (0, 0)),
        ],
        out_specs=pl.BlockSpec((_ROWS_BLK, H_F), lambda i: (i, 0)),
        out_shape=jax.ShapeDtypeStruct((N_NODES, H_F), jnp.float32),
    )(h, f1, aggf, aggf, dinv, g, b2r)


def kernel(features, edge_index, W1, b1, W2, b2):
    f32 = jnp.float32
    edges = edge_index.reshape(2, NCHUNKS, CHUNK)

    ones_deg = jnp.ones((CHUNK, DEG_W), f32)
    zeros_deg = jnp.zeros((RPS, DEG_W), f32)
    zeros_f = jnp.zeros((RPS, H_F), f32)

    w1t = W1.T
    b1r = b1.reshape(1, H_F)
    w2b = W2.T.reshape(D_POLY + 1, H_F, H_F)
    coeff = jnp.asarray(_COEFF, f32)
    g = jnp.tensordot(coeff, w2b, axes=((0,), (0,)))
    g = g.reshape((D_POLY + 1) * H_F, H_F)
    b2r = b2.reshape(1, H_F)

    sc_degree, sc_segsum = _sc_kernels()
    degp = sc_degree(edges, ones_deg, zeros_deg)
    h = _tc_matmul(features, w1t, b1r)
    z0f, dinv = _tc_scale(h, degp)
    agg1 = sc_segsum(z0f.reshape(_N_BLKS * _ROWS_BLK, H_F), edges, zeros_f)
    agg1f = agg1.reshape(NC * NPAD * H_F // 128, 128)
    f1, z1f = _tc_update(h, agg1f, dinv)
    agg2 = sc_segsum(z1f.reshape(_N_BLKS * _ROWS_BLK, H_F), edges, zeros_f)
    agg2f = agg2.reshape(NC * NPAD * H_F // 128, 128)
    return _tc_final(h, f1, agg2f, dinv, g, b2r)
